# baseline (speedup 1.0000x reference)
import functools

import jax
import jax.numpy as jnp
from jax import lax
from jax.experimental import pallas as pl
from jax.experimental.pallas import tpu as pltpu
from jax.experimental.pallas import tpu_sc as plsc

NF = 8000
NA = 2000
N = NF + NA
IN = 128
HID = 64
R = 3
E = 320000

NC = 2
NS = 16
NW = NC * NS
CHUNK = 128
NCHUNK = 80
GBYTES = CHUNK * HID * 4
EPW = NCHUNK * CHUNK
EPC = NS * EPW
ACC_ROWS = 2 * N + 96
DUMMY = 2 * N
RPW = ACC_ROWS // NS

_f32 = jnp.float32


def _sc_edge_body(yflat, src3, dst3, zrows, out, srcv, dstv, rows_a, acc, sga):
    cid = lax.axis_index("c")
    sid = lax.axis_index("s")
    wid = cid * NS + sid
    pltpu.sync_copy(src3.at[wid], srcv)
    pltpu.sync_copy(dst3.at[wid], dstv)
    r0 = sid * RPW
    pltpu.sync_copy(zrows, acc.at[pl.ds(r0, RPW)])
    plsc.subcore_barrier()

    def round_(j, carry):
        pltpu.async_copy(yflat.at[srcv.at[j]], rows_a, sga).wait()
        pltpu.sync_copy(rows_a, acc.at[dstv.at[j]], add=True)
        return carry

    lax.fori_loop(0, NCHUNK, round_, 0)
    plsc.subcore_barrier()
    pltpu.sync_copy(acc.at[pl.ds(r0, RPW)], out.at[cid, pl.ds(r0, RPW)])


def _sc_count_body(dst3, ones16, z16, out, dstv, onesv, cnt):
    cid = lax.axis_index("c")
    sid = lax.axis_index("s")
    wid = cid * NS + sid
    pltpu.sync_copy(dst3.at[wid], dstv)
    pltpu.sync_copy(ones16, onesv)
    r0 = sid * RPW
    pltpu.sync_copy(z16, cnt.at[pl.ds(r0, RPW)])
    plsc.subcore_barrier()

    def body(j, carry):
        pltpu.sync_copy(onesv, cnt.at[dstv.at[j]], add=True)
        return carry

    lax.fori_loop(0, NCHUNK, body, 0)
    plsc.subcore_barrier()
    pltpu.sync_copy(cnt.at[pl.ds(r0, RPW)], out.at[cid, pl.ds(r0, RPW)])


@functools.cache
def _sc_kernels():
    mesh = plsc.VectorSubcoreMesh(core_axis_name="c", subcore_axis_name="s",
                                  num_cores=NC, num_subcores=NS)
    params = pltpu.CompilerParams(use_tc_tiling_on_sc=False)
    edge = pl.kernel(
        _sc_edge_body,
        out_type=jax.ShapeDtypeStruct((NC, ACC_ROWS, HID), _f32),
        mesh=mesh,
        compiler_params=params,
        scratch_types=(
            [pltpu.VMEM((NCHUNK, CHUNK), jnp.int32)] * 2
            + [pltpu.VMEM((CHUNK, HID), _f32)]
            + [pltpu.VMEM_SHARED((ACC_ROWS, HID), _f32)]
            + [pltpu.SemaphoreType.DMA]
        ),
    )
    count = pl.kernel(
        _sc_count_body,
        out_type=jax.ShapeDtypeStruct((NC, ACC_ROWS, 16), _f32),
        mesh=mesh,
        compiler_params=params,
        scratch_types=[
            pltpu.VMEM((NCHUNK, CHUNK), jnp.int32),
            pltpu.VMEM((CHUNK, 16), _f32),
            pltpu.VMEM_SHARED((ACC_ROWS, 16), _f32),
        ],
    )
    return edge, count


def _sc_edge_kernel(yflat, src3, dst3, zrows):
    return _sc_kernels()[0](yflat, src3, dst3, zrows)


def _sc_count_kernel(dst3, ones16, z16):
    return _sc_kernels()[1](dst3, ones16, z16)


def _rel_weights(comp_ref, basis_ref):
    wfull = jnp.tensordot(comp_ref[...], basis_ref[...], axes=[[1], [0]],
                          preferred_element_type=_f32)
    return [wfull[r] for r in range(R)]


def _tc_enc_body(x_ref, wenc_ref, benc_ref, comp_ref, basis_ref, root_ref,
                 yrel_ref, yroot_ref):
    x0 = jnp.dot(x_ref[...], wenc_ref[0], preferred_element_type=_f32)
    x0 = jnp.maximum(x0 + benc_ref[0], 0.0)
    ws = _rel_weights(comp_ref, basis_ref)
    for r in range(R):
        yrel_ref[r] = jnp.dot(x0, ws[r], preferred_element_type=_f32)
    yroot_ref[...] = jnp.dot(x0, root_ref[...], preferred_element_type=_f32)


def _tc_encode_transform(xf, xa, wencs, bencs, comp, basis, root):
    x = jnp.concatenate([xf, xa], axis=0)
    bm = 1000
    nb = N // bm
    nf_blocks = NF // bm
    sel = lambda i: jnp.where(i < nf_blocks, 0, 1)
    return pl.pallas_call(
        _tc_enc_body,
        grid=(nb,),
        in_specs=[
            pl.BlockSpec((bm, IN), lambda i: (i, 0)),
            pl.BlockSpec((1, IN, HID), lambda i: (sel(i), 0, 0)),
            pl.BlockSpec((1, 1, HID), lambda i: (sel(i), 0, 0)),
            pl.BlockSpec(comp.shape, lambda i: (0, 0)),
            pl.BlockSpec(basis.shape, lambda i: (0, 0, 0)),
            pl.BlockSpec((HID, HID), lambda i: (0, 0)),
        ],
        out_specs=[
            pl.BlockSpec((R, bm, HID), lambda i: (0, i, 0)),
            pl.BlockSpec((bm, HID), lambda i: (i, 0)),
        ],
        out_shape=[
            jax.ShapeDtypeStruct((R, N, HID), _f32),
            jax.ShapeDtypeStruct((N, HID), _f32),
        ],
    )(x, wencs, bencs, comp, basis, root)


def _mean_relu(accp_ref, cntp_ref, yroot_ref, bias_ref):
    s = yroot_ref[...] + bias_ref[...]
    for c in range(NC):
        a = accp_ref[c, 0]
        cnt = jnp.max(cntp_ref[c, 0], axis=-1, keepdims=True)
        s = s + a * (1.0 / jnp.maximum(cnt, 1.0))
    a2 = accp_ref[0, 1] + accp_ref[1, 1]
    c2 = jnp.max(cntp_ref[0, 1] + cntp_ref[1, 1], axis=-1, keepdims=True)
    s = s + a2 * (1.0 / jnp.maximum(c2, 1.0))
    return jnp.maximum(s, 0.0)


def _tc_epi_mid_body(accp_ref, cntp_ref, yroot_ref, bias_ref, comp_ref,
                     basis_ref, root_ref, yrel_ref, yroot2_ref):
    x1 = _mean_relu(accp_ref, cntp_ref, yroot_ref, bias_ref)
    ws = _rel_weights(comp_ref, basis_ref)
    for r in range(R):
        yrel_ref[r] = jnp.dot(x1, ws[r], preferred_element_type=_f32)
    yroot2_ref[...] = jnp.dot(x1, root_ref[...], preferred_element_type=_f32)


def _tc_epilogue_mid(accp, cntp, yroot, bias, comp, basis, root):
    bm = 1000
    nb = N // bm
    return pl.pallas_call(
        _tc_epi_mid_body,
        grid=(nb,),
        in_specs=[
            pl.BlockSpec((NC, 2, bm, HID), lambda i: (0, 0, i, 0)),
            pl.BlockSpec((NC, 2, bm, 16), lambda i: (0, 0, i, 0)),
            pl.BlockSpec((bm, HID), lambda i: (i, 0)),
            pl.BlockSpec((1, HID), lambda i: (0, 0)),
            pl.BlockSpec(comp.shape, lambda i: (0, 0)),
            pl.BlockSpec(basis.shape, lambda i: (0, 0, 0)),
            pl.BlockSpec((HID, HID), lambda i: (0, 0)),
        ],
        out_specs=[
            pl.BlockSpec((R, bm, HID), lambda i: (0, i, 0)),
            pl.BlockSpec((bm, HID), lambda i: (i, 0)),
        ],
        out_shape=[
            jax.ShapeDtypeStruct((R, N, HID), _f32),
            jax.ShapeDtypeStruct((N, HID), _f32),
        ],
    )(accp, cntp, yroot, bias.reshape(1, HID), comp, basis, root)


def _tc_epi_fin_body(accp_ref, cntp_ref, yroot_ref, bias_ref, wro_ref,
                     bro_ref, out_ref):
    x2 = _mean_relu(accp_ref, cntp_ref, yroot_ref, bias_ref)
    out_ref[...] = jnp.dot(x2, wro_ref[...], preferred_element_type=_f32) \
        + bro_ref[...]


def _tc_epilogue_final(accp, cntp, yroot, bias, wro, bro):
    bm = 1000
    nb = NF // bm
    return pl.pallas_call(
        _tc_epi_fin_body,
        grid=(nb,),
        in_specs=[
            pl.BlockSpec((NC, 2, bm, HID), lambda i: (0, 0, i, 0)),
            pl.BlockSpec((NC, 2, bm, 16), lambda i: (0, 0, i, 0)),
            pl.BlockSpec((bm, HID), lambda i: (i, 0)),
            pl.BlockSpec((1, HID), lambda i: (0, 0)),
            pl.BlockSpec((HID, 1), lambda i: (0, 0)),
            pl.BlockSpec((1, 1), lambda i: (0, 0)),
        ],
        out_specs=pl.BlockSpec((bm, 1), lambda i: (i, 0)),
        out_shape=jax.ShapeDtypeStruct((NF, 1), _f32),
    )(accp, cntp, yroot, bias.reshape(1, HID), wro, bro.reshape(1, 1))


def kernel(x_flight, x_airport, edge_index_fa, edge_index_af, edge_index_ff,
           W_enc_flight, b_enc_flight, W_enc_airport, b_enc_airport,
           comp0, basis0, root0, bias0, comp1, basis1, root1, bias1,
           W_ro, b_ro):
    fa0 = edge_index_fa[0].astype(jnp.int32)
    fa1 = edge_index_fa[1].astype(jnp.int32)
    af0 = edge_index_af[0].astype(jnp.int32)
    af1 = edge_index_af[1].astype(jnp.int32)
    ff0 = edge_index_ff[0].astype(jnp.int32)
    ff1 = edge_index_ff[1].astype(jnp.int32)
    eh = ff0.shape[0] // 2
    npad = EPC - (fa0.shape[0] + eh)
    padi = jnp.zeros((npad,), jnp.int32)
    padd = jnp.full((npad,), DUMMY, jnp.int32)
    src_c0 = jnp.concatenate([fa0, ff0[:eh] + 2 * N, padi])
    dst_c0 = jnp.concatenate([fa1 + NF, ff1[:eh] + N, padd])
    src_c1 = jnp.concatenate([af0 + (NF + N), ff0[eh:] + 2 * N, padi])
    dst_c1 = jnp.concatenate([af1, ff1[eh:] + N, padd])
    src3 = jnp.concatenate([src_c0, src_c1]).reshape(NW, NCHUNK, CHUNK)
    dst3 = jnp.concatenate([dst_c0, dst_c1]).reshape(NW, NCHUNK, CHUNK)

    zrows = jnp.zeros((RPW, HID), _f32)
    z16 = jnp.zeros((RPW, 16), _f32)
    ones16 = jnp.ones((CHUNK, 16), _f32)

    cnt_out = _sc_count_kernel(dst3, ones16, z16)
    cntp = cnt_out[:, :2 * N, :].reshape(NC, 2, N, 16)

    wencs = jnp.stack([W_enc_flight, W_enc_airport])
    bencs = jnp.stack([b_enc_flight, b_enc_airport]).reshape(2, 1, HID)
    yrel0, yroot0 = _tc_encode_transform(
        x_flight, x_airport, wencs, bencs, comp0, basis0, root0)
    yflat0 = yrel0.reshape(R * N, HID)

    acc0 = _sc_edge_kernel(yflat0, src3, dst3, zrows)
    accp0 = acc0[:, :2 * N, :].reshape(NC, 2, N, HID)
    yrel1, yroot1 = _tc_epilogue_mid(
        accp0, cntp, yroot0, bias0, comp1, basis1, root1)
    yflat1 = yrel1.reshape(R * N, HID)

    acc1 = _sc_edge_kernel(yflat1, src3, dst3, zrows)
    accp1 = acc1[:, :2 * N, :].reshape(NC, 2, N, HID)
    out = _tc_epilogue_final(accp1, cntp, yroot1[:NF], bias1, W_ro, b_ro)
    return out.squeeze(-1)


# speedup vs baseline: 1.2075x; 1.2075x over previous
import functools

import jax
import jax.numpy as jnp
from jax import lax
from jax.experimental import pallas as pl
from jax.experimental.pallas import tpu as pltpu
from jax.experimental.pallas import tpu_sc as plsc

NF = 8000
NA = 2000
N = NF + NA
IN = 128
HID = 64
R = 3
E = 320000

NC = 2
NS = 16
NW = NC * NS
CHUNK = 128
NCHUNK = 79
GBYTES = CHUNK * HID * 4
EPW = NCHUNK * CHUNK
EPC = NS * EPW
ACC_ROWS = 2 * N + 96
DUMMY = 2 * N
RPW = ACC_ROWS // NS

_f32 = jnp.float32


def _sc_edge_body(yflat, src3, dst3, zrows, out, srcv, dstv, rows_a, acc, sga):
    cid = lax.axis_index("c")
    sid = lax.axis_index("s")
    wid = cid * NS + sid
    pltpu.sync_copy(src3.at[wid], srcv)
    pltpu.sync_copy(dst3.at[wid], dstv)
    r0 = sid * RPW
    pltpu.sync_copy(zrows, acc.at[pl.ds(r0, RPW)])
    plsc.subcore_barrier()

    def round_(j, carry):
        pltpu.async_copy(yflat.at[srcv.at[j]], rows_a, sga).wait()
        pltpu.sync_copy(rows_a, acc.at[dstv.at[j]], add=True)
        return carry

    lax.fori_loop(0, NCHUNK, round_, 0)
    plsc.subcore_barrier()
    pltpu.sync_copy(acc.at[pl.ds(r0, RPW)], out.at[cid, pl.ds(r0, RPW)])


def _sc_count_body(dst3, ones16, z16, out, dstv, onesv, cnt):
    cid = lax.axis_index("c")
    sid = lax.axis_index("s")
    wid = cid * NS + sid
    pltpu.sync_copy(dst3.at[wid], dstv)
    pltpu.sync_copy(ones16, onesv)
    r0 = sid * RPW
    pltpu.sync_copy(z16, cnt.at[pl.ds(r0, RPW)])
    plsc.subcore_barrier()

    def body(j, carry):
        pltpu.sync_copy(onesv, cnt.at[dstv.at[j]], add=True)
        return carry

    lax.fori_loop(0, NCHUNK, body, 0)
    plsc.subcore_barrier()
    pltpu.sync_copy(cnt.at[pl.ds(r0, RPW)], out.at[cid, pl.ds(r0, RPW)])


@functools.cache
def _sc_kernels():
    mesh = plsc.VectorSubcoreMesh(core_axis_name="c", subcore_axis_name="s",
                                  num_cores=NC, num_subcores=NS)
    params = pltpu.CompilerParams(use_tc_tiling_on_sc=False)
    edge = pl.kernel(
        _sc_edge_body,
        out_type=jax.ShapeDtypeStruct((NC, ACC_ROWS, HID), _f32),
        mesh=mesh,
        compiler_params=params,
        scratch_types=(
            [pltpu.VMEM((NCHUNK, CHUNK), jnp.int32)] * 2
            + [pltpu.VMEM((CHUNK, HID), _f32)]
            + [pltpu.VMEM_SHARED((ACC_ROWS, HID), _f32)]
            + [pltpu.SemaphoreType.DMA]
        ),
    )
    count = pl.kernel(
        _sc_count_body,
        out_type=jax.ShapeDtypeStruct((NC, ACC_ROWS, 16), _f32),
        mesh=mesh,
        compiler_params=params,
        scratch_types=[
            pltpu.VMEM((NCHUNK, CHUNK), jnp.int32),
            pltpu.VMEM((CHUNK, 16), _f32),
            pltpu.VMEM_SHARED((ACC_ROWS, 16), _f32),
        ],
    )
    return edge, count


def _sc_edge_kernel(yflat, src3, dst3, zrows):
    return _sc_kernels()[0](yflat, src3, dst3, zrows)


def _sc_count_kernel(dst3, ones16, z16):
    return _sc_kernels()[1](dst3, ones16, z16)


def _rel_weights(comp_ref, basis_ref):
    wfull = jnp.tensordot(comp_ref[...], basis_ref[...], axes=[[1], [0]],
                          preferred_element_type=_f32)
    return [wfull[r] for r in range(R)]


def _tc_enc_body(x_ref, wenc_ref, benc_ref, comp_ref, basis_ref, root_ref,
                 yrel_ref, yroot_ref):
    x0 = jnp.dot(x_ref[...], wenc_ref[0], preferred_element_type=_f32)
    x0 = jnp.maximum(x0 + benc_ref[0], 0.0)
    ws = _rel_weights(comp_ref, basis_ref)
    for r in range(R):
        yrel_ref[r] = jnp.dot(x0, ws[r], preferred_element_type=_f32)
    yroot_ref[...] = jnp.dot(x0, root_ref[...], preferred_element_type=_f32)


def _tc_encode_transform(x, wenc, benc, comp, basis, root):
    rows = x.shape[0]
    bm = 1000
    nb = rows // bm
    return pl.pallas_call(
        _tc_enc_body,
        grid=(nb,),
        in_specs=[
            pl.BlockSpec((bm, IN), lambda i: (i, 0)),
            pl.BlockSpec((1, IN, HID), lambda i: (0, 0, 0)),
            pl.BlockSpec((1, 1, HID), lambda i: (0, 0, 0)),
            pl.BlockSpec(comp.shape, lambda i: (0, 0)),
            pl.BlockSpec(basis.shape, lambda i: (0, 0, 0)),
            pl.BlockSpec((HID, HID), lambda i: (0, 0)),
        ],
        out_specs=[
            pl.BlockSpec((R, bm, HID), lambda i: (0, i, 0)),
            pl.BlockSpec((bm, HID), lambda i: (i, 0)),
        ],
        out_shape=[
            jax.ShapeDtypeStruct((R, rows, HID), _f32),
            jax.ShapeDtypeStruct((rows, HID), _f32),
        ],
    )(x, wenc, benc, comp, basis, root)


def _mean_relu(accp_ref, cntp_ref, yroot_ref, bias_ref):
    s = yroot_ref[...] + bias_ref[...]
    for c in range(NC):
        a = accp_ref[c, 0]
        cnt = jnp.max(cntp_ref[c, 0], axis=-1, keepdims=True)
        s = s + a * (1.0 / jnp.maximum(cnt, 1.0))
    a2 = accp_ref[0, 1] + accp_ref[1, 1]
    c2 = jnp.max(cntp_ref[0, 1] + cntp_ref[1, 1], axis=-1, keepdims=True)
    s = s + a2 * (1.0 / jnp.maximum(c2, 1.0))
    return jnp.maximum(s, 0.0)


def _tc_epi_mid_body(accp_ref, cntp_ref, yroot_ref, bias_ref, comp_ref,
                     basis_ref, root_ref, yrel_ref, yroot2_ref):
    x1 = _mean_relu(accp_ref, cntp_ref, yroot_ref, bias_ref)
    ws = _rel_weights(comp_ref, basis_ref)
    for r in range(R):
        yrel_ref[r] = jnp.dot(x1, ws[r], preferred_element_type=_f32)
    yroot2_ref[...] = jnp.dot(x1, root_ref[...], preferred_element_type=_f32)


def _tc_epilogue_mid(accp, cntp, yroot, bias, comp, basis, root):
    bm = 1000
    nb = N // bm
    return pl.pallas_call(
        _tc_epi_mid_body,
        grid=(nb,),
        in_specs=[
            pl.BlockSpec((NC, 2, bm, HID), lambda i: (0, 0, i, 0)),
            pl.BlockSpec((NC, 2, bm, 16), lambda i: (0, 0, i, 0)),
            pl.BlockSpec((bm, HID), lambda i: (i, 0)),
            pl.BlockSpec((1, HID), lambda i: (0, 0)),
            pl.BlockSpec(comp.shape, lambda i: (0, 0)),
            pl.BlockSpec(basis.shape, lambda i: (0, 0, 0)),
            pl.BlockSpec((HID, HID), lambda i: (0, 0)),
        ],
        out_specs=[
            pl.BlockSpec((R, bm, HID), lambda i: (0, i, 0)),
            pl.BlockSpec((bm, HID), lambda i: (i, 0)),
        ],
        out_shape=[
            jax.ShapeDtypeStruct((R, N, HID), _f32),
            jax.ShapeDtypeStruct((N, HID), _f32),
        ],
    )(accp, cntp, yroot, bias.reshape(1, HID), comp, basis, root)


def _tc_epi_fin_body(accp_ref, cntp_ref, yroot_ref, bias_ref, wro_ref,
                     bro_ref, out_ref):
    x2 = _mean_relu(accp_ref, cntp_ref, yroot_ref, bias_ref)
    out_ref[...] = jnp.dot(x2, wro_ref[...], preferred_element_type=_f32) \
        + bro_ref[...]


def _tc_epilogue_final(accp, cntp, yroot, bias, wro, bro):
    bm = 1000
    nb = NF // bm
    return pl.pallas_call(
        _tc_epi_fin_body,
        grid=(nb,),
        in_specs=[
            pl.BlockSpec((NC, 2, bm, HID), lambda i: (0, 0, i, 0)),
            pl.BlockSpec((NC, 2, bm, 16), lambda i: (0, 0, i, 0)),
            pl.BlockSpec((bm, HID), lambda i: (i, 0)),
            pl.BlockSpec((1, HID), lambda i: (0, 0)),
            pl.BlockSpec((HID, 1), lambda i: (0, 0)),
            pl.BlockSpec((1, 1), lambda i: (0, 0)),
        ],
        out_specs=pl.BlockSpec((bm, 1), lambda i: (i, 0)),
        out_shape=jax.ShapeDtypeStruct((NF, 1), _f32),
    )(accp, cntp, yroot, bias.reshape(1, HID), wro, bro.reshape(1, 1))


def kernel(x_flight, x_airport, edge_index_fa, edge_index_af, edge_index_ff,
           W_enc_flight, b_enc_flight, W_enc_airport, b_enc_airport,
           comp0, basis0, root0, bias0, comp1, basis1, root1, bias1,
           W_ro, b_ro):
    fa0 = edge_index_fa[0].astype(jnp.int32)
    fa1 = edge_index_fa[1].astype(jnp.int32)
    af0 = edge_index_af[0].astype(jnp.int32)
    af1 = edge_index_af[1].astype(jnp.int32)
    ff0 = edge_index_ff[0].astype(jnp.int32)
    ff1 = edge_index_ff[1].astype(jnp.int32)
    eh = ff0.shape[0] // 2
    npad = EPC - (fa0.shape[0] + eh)
    padi = jnp.zeros((npad,), jnp.int32)
    padd = jnp.full((npad,), DUMMY, jnp.int32)
    src_c0 = jnp.concatenate([fa0, ff0[:eh] + 2 * N, padi])
    dst_c0 = jnp.concatenate([fa1 + NF, ff1[:eh] + N, padd])
    src_c1 = jnp.concatenate([af0 + (NF + N), ff0[eh:] + 2 * N, padi])
    dst_c1 = jnp.concatenate([af1, ff1[eh:] + N, padd])
    src3 = jnp.concatenate([src_c0, src_c1]).reshape(NW, NCHUNK, CHUNK)
    dst3 = jnp.concatenate([dst_c0, dst_c1]).reshape(NW, NCHUNK, CHUNK)

    zrows = jnp.zeros((RPW, HID), _f32)
    z16 = jnp.zeros((RPW, 16), _f32)
    ones16 = jnp.ones((CHUNK, 16), _f32)

    cnt_out = _sc_count_kernel(dst3, ones16, z16)
    cntp = cnt_out[:, :2 * N, :].reshape(NC, 2, N, 16)

    yrel_f, yroot_f = _tc_encode_transform(
        x_flight, W_enc_flight.reshape(1, IN, HID),
        b_enc_flight.reshape(1, 1, HID), comp0, basis0, root0)
    yrel_a, yroot_a = _tc_encode_transform(
        x_airport, W_enc_airport.reshape(1, IN, HID),
        b_enc_airport.reshape(1, 1, HID), comp0, basis0, root0)
    yrel0 = jnp.concatenate([yrel_f, yrel_a], axis=1)
    yroot0 = jnp.concatenate([yroot_f, yroot_a], axis=0)
    yflat0 = yrel0.reshape(R * N, HID)

    acc0 = _sc_edge_kernel(yflat0, src3, dst3, zrows)
    accp0 = acc0[:, :2 * N, :].reshape(NC, 2, N, HID)
    yrel1, yroot1 = _tc_epilogue_mid(
        accp0, cntp, yroot0, bias0, comp1, basis1, root1)
    yflat1 = yrel1.reshape(R * N, HID)

    acc1 = _sc_edge_kernel(yflat1, src3, dst3, zrows)
    accp1 = acc1[:, :2 * N, :].reshape(NC, 2, N, HID)
    out = _tc_epilogue_final(accp1, cntp, yroot1[:NF], bias1, W_ro, b_ro)
    return out.squeeze(-1)


# speedup vs baseline: 1.3055x; 1.0812x over previous
import functools

import jax
import jax.numpy as jnp
from jax import lax
from jax.experimental import pallas as pl
from jax.experimental.pallas import tpu as pltpu
from jax.experimental.pallas import tpu_sc as plsc

NF = 8000
NA = 2000
N = NF + NA
IN = 128
HID = 64
R = 3
E = 320000

NC = 2
NS = 16
NW = NC * NS
CHUNK = 128
NCHUNK = 79
GBYTES = CHUNK * HID * 4
EPW = NCHUNK * CHUNK
EPC = NS * EPW
ACC_ROWS = 2 * N + 96
DUMMY = 2 * N
RPW = ACC_ROWS // NS

_f32 = jnp.float32


def _sc_edge_body(yflat, src3, dst3, zrows, out, srcv, dstv, rows_a, rows_b,
                  acc, sga, sgb, ssa, ssb):
    cid = lax.axis_index("c")
    sid = lax.axis_index("s")
    wid = cid * NS + sid
    pltpu.sync_copy(src3.at[wid], srcv)
    pltpu.sync_copy(dst3.at[wid], dstv)
    r0 = sid * RPW
    pltpu.sync_copy(zrows, acc.at[pl.ds(r0, RPW)])
    plsc.subcore_barrier()

    dummy = yflat.at[pl.ds(0, CHUNK)]
    drain_ga = pltpu.make_async_copy(dummy, rows_a, sga)
    drain_gb = pltpu.make_async_copy(dummy, rows_b, sgb)
    drain_sa = pltpu.make_async_copy(dummy, rows_a, ssa)
    drain_sb = pltpu.make_async_copy(dummy, rows_b, ssb)
    pltpu.async_copy(yflat.at[srcv.at[0]], rows_a, sga)

    def round_(k, carry):
        j0 = 2 * k
        j1 = j0 + 1
        drain_ga.wait()
        pltpu.async_copy(yflat.at[srcv.at[j1]], rows_b, sgb)
        pltpu.async_copy(rows_a, acc.at[dstv.at[j0]], ssa, add=True)
        drain_gb.wait()
        drain_sa.wait()
        pltpu.async_copy(yflat.at[srcv.at[j0 + 2]], rows_a, sga)
        pltpu.async_copy(rows_b, acc.at[dstv.at[j1]], ssb, add=True)
        drain_sb.wait()
        return carry

    lax.fori_loop(0, (NCHUNK - 1) // 2, round_, 0)
    drain_ga.wait()
    pltpu.sync_copy(rows_a, acc.at[dstv.at[NCHUNK - 1]], add=True)
    plsc.subcore_barrier()
    pltpu.sync_copy(acc.at[pl.ds(r0, RPW)], out.at[cid, pl.ds(r0, RPW)])


def _sc_count_body(dst3, ones16, z16, out, dstv, onesv, cnt):
    cid = lax.axis_index("c")
    sid = lax.axis_index("s")
    wid = cid * NS + sid
    pltpu.sync_copy(dst3.at[wid], dstv)
    pltpu.sync_copy(ones16, onesv)
    r0 = sid * RPW
    pltpu.sync_copy(z16, cnt.at[pl.ds(r0, RPW)])
    plsc.subcore_barrier()

    def body(j, carry):
        pltpu.sync_copy(onesv, cnt.at[dstv.at[j]], add=True)
        return carry

    lax.fori_loop(0, NCHUNK, body, 0)
    plsc.subcore_barrier()
    pltpu.sync_copy(cnt.at[pl.ds(r0, RPW)], out.at[cid, pl.ds(r0, RPW)])


@functools.cache
def _sc_kernels():
    mesh = plsc.VectorSubcoreMesh(core_axis_name="c", subcore_axis_name="s",
                                  num_cores=NC, num_subcores=NS)
    params = pltpu.CompilerParams(use_tc_tiling_on_sc=False)
    edge = pl.kernel(
        _sc_edge_body,
        out_type=jax.ShapeDtypeStruct((NC, ACC_ROWS, HID), _f32),
        mesh=mesh,
        compiler_params=params,
        scratch_types=(
            [pltpu.VMEM((NCHUNK, CHUNK), jnp.int32)] * 2
            + [pltpu.VMEM((CHUNK, HID), _f32)] * 2
            + [pltpu.VMEM_SHARED((ACC_ROWS, HID), _f32)]
            + [pltpu.SemaphoreType.DMA] * 4
        ),
    )
    count = pl.kernel(
        _sc_count_body,
        out_type=jax.ShapeDtypeStruct((NC, ACC_ROWS, 16), _f32),
        mesh=mesh,
        compiler_params=params,
        scratch_types=[
            pltpu.VMEM((NCHUNK, CHUNK), jnp.int32),
            pltpu.VMEM((CHUNK, 16), _f32),
            pltpu.VMEM_SHARED((ACC_ROWS, 16), _f32),
        ],
    )
    return edge, count


def _sc_edge_kernel(yflat, src3, dst3, zrows):
    return _sc_kernels()[0](yflat, src3, dst3, zrows)


def _sc_count_kernel(dst3, ones16, z16):
    return _sc_kernels()[1](dst3, ones16, z16)


def _rel_weights(comp_ref, basis_ref):
    wfull = jnp.tensordot(comp_ref[...], basis_ref[...], axes=[[1], [0]],
                          preferred_element_type=_f32)
    return [wfull[r] for r in range(R)]


def _tc_enc_body(x_ref, wenc_ref, benc_ref, comp_ref, basis_ref, root_ref,
                 yrel_ref, yroot_ref):
    x0 = jnp.dot(x_ref[...], wenc_ref[0], preferred_element_type=_f32)
    x0 = jnp.maximum(x0 + benc_ref[0], 0.0)
    ws = _rel_weights(comp_ref, basis_ref)
    for r in range(R):
        yrel_ref[r] = jnp.dot(x0, ws[r], preferred_element_type=_f32)
    yroot_ref[...] = jnp.dot(x0, root_ref[...], preferred_element_type=_f32)


def _tc_encode_transform(x, wenc, benc, comp, basis, root):
    rows = x.shape[0]
    bm = 1000
    nb = rows // bm
    return pl.pallas_call(
        _tc_enc_body,
        grid=(nb,),
        in_specs=[
            pl.BlockSpec((bm, IN), lambda i: (i, 0)),
            pl.BlockSpec((1, IN, HID), lambda i: (0, 0, 0)),
            pl.BlockSpec((1, 1, HID), lambda i: (0, 0, 0)),
            pl.BlockSpec(comp.shape, lambda i: (0, 0)),
            pl.BlockSpec(basis.shape, lambda i: (0, 0, 0)),
            pl.BlockSpec((HID, HID), lambda i: (0, 0)),
        ],
        out_specs=[
            pl.BlockSpec((R, bm, HID), lambda i: (0, i, 0)),
            pl.BlockSpec((bm, HID), lambda i: (i, 0)),
        ],
        out_shape=[
            jax.ShapeDtypeStruct((R, rows, HID), _f32),
            jax.ShapeDtypeStruct((rows, HID), _f32),
        ],
    )(x, wenc, benc, comp, basis, root)


def _mean_relu(accp_ref, cntp_ref, yroot_ref, bias_ref):
    s = yroot_ref[...] + bias_ref[...]
    for c in range(NC):
        a = accp_ref[c, 0]
        cnt = jnp.max(cntp_ref[c, 0], axis=-1, keepdims=True)
        s = s + a * (1.0 / jnp.maximum(cnt, 1.0))
    a2 = accp_ref[0, 1] + accp_ref[1, 1]
    c2 = jnp.max(cntp_ref[0, 1] + cntp_ref[1, 1], axis=-1, keepdims=True)
    s = s + a2 * (1.0 / jnp.maximum(c2, 1.0))
    return jnp.maximum(s, 0.0)


def _tc_epi_mid_body(accp_ref, cntp_ref, yroot_ref, bias_ref, comp_ref,
                     basis_ref, root_ref, yrel_ref, yroot2_ref):
    x1 = _mean_relu(accp_ref, cntp_ref, yroot_ref, bias_ref)
    ws = _rel_weights(comp_ref, basis_ref)
    for r in range(R):
        yrel_ref[r] = jnp.dot(x1, ws[r], preferred_element_type=_f32)
    yroot2_ref[...] = jnp.dot(x1, root_ref[...], preferred_element_type=_f32)


def _tc_epilogue_mid(accp, cntp, yroot, bias, comp, basis, root):
    bm = 1000
    nb = N // bm
    return pl.pallas_call(
        _tc_epi_mid_body,
        grid=(nb,),
        in_specs=[
            pl.BlockSpec((NC, 2, bm, HID), lambda i: (0, 0, i, 0)),
            pl.BlockSpec((NC, 2, bm, 16), lambda i: (0, 0, i, 0)),
            pl.BlockSpec((bm, HID), lambda i: (i, 0)),
            pl.BlockSpec((1, HID), lambda i: (0, 0)),
            pl.BlockSpec(comp.shape, lambda i: (0, 0)),
            pl.BlockSpec(basis.shape, lambda i: (0, 0, 0)),
            pl.BlockSpec((HID, HID), lambda i: (0, 0)),
        ],
        out_specs=[
            pl.BlockSpec((R, bm, HID), lambda i: (0, i, 0)),
            pl.BlockSpec((bm, HID), lambda i: (i, 0)),
        ],
        out_shape=[
            jax.ShapeDtypeStruct((R, N, HID), _f32),
            jax.ShapeDtypeStruct((N, HID), _f32),
        ],
    )(accp, cntp, yroot, bias.reshape(1, HID), comp, basis, root)


def _tc_epi_fin_body(accp_ref, cntp_ref, yroot_ref, bias_ref, wro_ref,
                     bro_ref, out_ref):
    x2 = _mean_relu(accp_ref, cntp_ref, yroot_ref, bias_ref)
    out_ref[...] = jnp.dot(x2, wro_ref[...], preferred_element_type=_f32) \
        + bro_ref[...]


def _tc_epilogue_final(accp, cntp, yroot, bias, wro, bro):
    bm = 1000
    nb = NF // bm
    return pl.pallas_call(
        _tc_epi_fin_body,
        grid=(nb,),
        in_specs=[
            pl.BlockSpec((NC, 2, bm, HID), lambda i: (0, 0, i, 0)),
            pl.BlockSpec((NC, 2, bm, 16), lambda i: (0, 0, i, 0)),
            pl.BlockSpec((bm, HID), lambda i: (i, 0)),
            pl.BlockSpec((1, HID), lambda i: (0, 0)),
            pl.BlockSpec((HID, 1), lambda i: (0, 0)),
            pl.BlockSpec((1, 1), lambda i: (0, 0)),
        ],
        out_specs=pl.BlockSpec((bm, 1), lambda i: (i, 0)),
        out_shape=jax.ShapeDtypeStruct((NF, 1), _f32),
    )(accp, cntp, yroot, bias.reshape(1, HID), wro, bro.reshape(1, 1))


def kernel(x_flight, x_airport, edge_index_fa, edge_index_af, edge_index_ff,
           W_enc_flight, b_enc_flight, W_enc_airport, b_enc_airport,
           comp0, basis0, root0, bias0, comp1, basis1, root1, bias1,
           W_ro, b_ro):
    fa0 = edge_index_fa[0].astype(jnp.int32)
    fa1 = edge_index_fa[1].astype(jnp.int32)
    af0 = edge_index_af[0].astype(jnp.int32)
    af1 = edge_index_af[1].astype(jnp.int32)
    ff0 = edge_index_ff[0].astype(jnp.int32)
    ff1 = edge_index_ff[1].astype(jnp.int32)
    eh = ff0.shape[0] // 2
    npad = EPC - (fa0.shape[0] + eh)
    padi = jnp.zeros((npad,), jnp.int32)
    padd = jnp.full((npad,), DUMMY, jnp.int32)
    src_c0 = jnp.concatenate([fa0, ff0[:eh] + 2 * N, padi])
    dst_c0 = jnp.concatenate([fa1 + NF, ff1[:eh] + N, padd])
    src_c1 = jnp.concatenate([af0 + (NF + N), ff0[eh:] + 2 * N, padi])
    dst_c1 = jnp.concatenate([af1, ff1[eh:] + N, padd])
    src3 = jnp.concatenate([src_c0, src_c1]).reshape(NW, NCHUNK, CHUNK)
    dst3 = jnp.concatenate([dst_c0, dst_c1]).reshape(NW, NCHUNK, CHUNK)

    zrows = jnp.zeros((RPW, HID), _f32)
    z16 = jnp.zeros((RPW, 16), _f32)
    ones16 = jnp.ones((CHUNK, 16), _f32)

    cnt_out = _sc_count_kernel(dst3, ones16, z16)
    cntp = cnt_out[:, :2 * N, :].reshape(NC, 2, N, 16)

    yrel_f, yroot_f = _tc_encode_transform(
        x_flight, W_enc_flight.reshape(1, IN, HID),
        b_enc_flight.reshape(1, 1, HID), comp0, basis0, root0)
    yrel_a, yroot_a = _tc_encode_transform(
        x_airport, W_enc_airport.reshape(1, IN, HID),
        b_enc_airport.reshape(1, 1, HID), comp0, basis0, root0)
    yrel0 = jnp.concatenate([yrel_f, yrel_a], axis=1)
    yroot0 = jnp.concatenate([yroot_f, yroot_a], axis=0)
    yflat0 = yrel0.reshape(R * N, HID)

    acc0 = _sc_edge_kernel(yflat0, src3, dst3, zrows)
    accp0 = acc0[:, :2 * N, :].reshape(NC, 2, N, HID)
    yrel1, yroot1 = _tc_epilogue_mid(
        accp0, cntp, yroot0, bias0, comp1, basis1, root1)
    yflat1 = yrel1.reshape(R * N, HID)

    acc1 = _sc_edge_kernel(yflat1, src3, dst3, zrows)
    accp1 = acc1[:, :2 * N, :].reshape(NC, 2, N, HID)
    out = _tc_epilogue_final(accp1, cntp, yroot1[:NF], bias1, W_ro, b_ro)
    return out.squeeze(-1)


# speedup vs baseline: 1.4322x; 1.0970x over previous
import functools

import jax
import jax.numpy as jnp
from jax import lax
from jax.experimental import pallas as pl
from jax.experimental.pallas import tpu as pltpu
from jax.experimental.pallas import tpu_sc as plsc

NF = 8000
NA = 2000
N = NF + NA
IN = 128
HID = 64
R = 3
E = 320000

NC = 2
NS = 16
NW = NC * NS
CHUNK = 128
NCHUNK = 79
GBYTES = CHUNK * HID * 4
EPW = NCHUNK * CHUNK
EPC = NS * EPW
ACC_ROWS = 2 * N + 96
DUMMY = 2 * N
RPW = ACC_ROWS // NS

_f32 = jnp.float32


def _sc_edge_body(yflat, src3, dst3, zrows, out, srcv, dstv,
                  rows_a, rows_b, rows_c,
                  acc, sga, sgb, sgc, ssa, ssb, ssc):
    cid = lax.axis_index("c")
    sid = lax.axis_index("s")
    wid = cid * NS + sid
    pltpu.sync_copy(src3.at[wid], srcv)
    pltpu.sync_copy(dst3.at[wid], dstv)
    r0 = sid * RPW
    pltpu.sync_copy(zrows, acc.at[pl.ds(r0, RPW)])
    plsc.subcore_barrier()

    dummy = yflat.at[pl.ds(0, CHUNK)]
    drain_ga = pltpu.make_async_copy(dummy, rows_a, sga)
    drain_gb = pltpu.make_async_copy(dummy, rows_b, sgb)
    drain_gc = pltpu.make_async_copy(dummy, rows_c, sgc)
    drain_sa = pltpu.make_async_copy(dummy, rows_a, ssa)
    drain_sb = pltpu.make_async_copy(dummy, rows_b, ssb)
    drain_sc = pltpu.make_async_copy(dummy, rows_c, ssc)
    pltpu.async_copy(yflat.at[srcv.at[0]], rows_a, sga)
    pltpu.async_copy(yflat.at[srcv.at[1]], rows_b, sgb)
    nrounds = (NCHUNK - 1) // 3

    def round_(k, carry):
        j0 = 3 * k
        drain_ga.wait()
        pltpu.async_copy(yflat.at[srcv.at[j0 + 2]], rows_c, sgc)
        pltpu.async_copy(rows_a, acc.at[dstv.at[j0]], ssa, add=True)
        drain_gb.wait()
        drain_sa.wait()
        pltpu.async_copy(yflat.at[srcv.at[j0 + 3]], rows_a, sga)
        pltpu.async_copy(rows_b, acc.at[dstv.at[j0 + 1]], ssb, add=True)
        drain_gc.wait()
        drain_sb.wait()

        @pl.when(k + 1 < nrounds)
        def _():
            pltpu.async_copy(yflat.at[srcv.at[j0 + 4]], rows_b, sgb)

        pltpu.async_copy(rows_c, acc.at[dstv.at[j0 + 2]], ssc, add=True)
        drain_sc.wait()
        return carry

    lax.fori_loop(0, nrounds, round_, 0)
    drain_ga.wait()
    pltpu.sync_copy(rows_a, acc.at[dstv.at[NCHUNK - 1]], add=True)
    plsc.subcore_barrier()
    pltpu.sync_copy(acc.at[pl.ds(r0, RPW)], out.at[cid, pl.ds(r0, RPW)])


def _sc_count_body(dst3, ones16, z16, out, dstv, onesv, cnt):
    cid = lax.axis_index("c")
    sid = lax.axis_index("s")
    wid = cid * NS + sid
    pltpu.sync_copy(dst3.at[wid], dstv)
    pltpu.sync_copy(ones16, onesv)
    r0 = sid * RPW
    pltpu.sync_copy(z16, cnt.at[pl.ds(r0, RPW)])
    plsc.subcore_barrier()

    def body(j, carry):
        pltpu.sync_copy(onesv, cnt.at[dstv.at[j]], add=True)
        return carry

    lax.fori_loop(0, NCHUNK, body, 0)
    plsc.subcore_barrier()
    pltpu.sync_copy(cnt.at[pl.ds(r0, RPW)], out.at[cid, pl.ds(r0, RPW)])


@functools.cache
def _sc_kernels():
    mesh = plsc.VectorSubcoreMesh(core_axis_name="c", subcore_axis_name="s",
                                  num_cores=NC, num_subcores=NS)
    params = pltpu.CompilerParams(use_tc_tiling_on_sc=False)
    edge = pl.kernel(
        _sc_edge_body,
        out_type=jax.ShapeDtypeStruct((NC, ACC_ROWS, HID), _f32),
        mesh=mesh,
        compiler_params=params,
        scratch_types=(
            [pltpu.VMEM((NCHUNK, CHUNK), jnp.int32)] * 2
            + [pltpu.VMEM((CHUNK, HID), _f32)] * 3
            + [pltpu.VMEM_SHARED((ACC_ROWS, HID), _f32)]
            + [pltpu.SemaphoreType.DMA] * 6
        ),
    )
    count = pl.kernel(
        _sc_count_body,
        out_type=jax.ShapeDtypeStruct((NC, ACC_ROWS, 16), _f32),
        mesh=mesh,
        compiler_params=params,
        scratch_types=[
            pltpu.VMEM((NCHUNK, CHUNK), jnp.int32),
            pltpu.VMEM((CHUNK, 16), _f32),
            pltpu.VMEM_SHARED((ACC_ROWS, 16), _f32),
        ],
    )
    return edge, count


def _sc_edge_kernel(yflat, src3, dst3, zrows):
    return _sc_kernels()[0](yflat, src3, dst3, zrows)


def _sc_count_kernel(dst3, ones16, z16):
    return _sc_kernels()[1](dst3, ones16, z16)


def _rel_weights(comp_ref, basis_ref):
    wfull = jnp.tensordot(comp_ref[...], basis_ref[...], axes=[[1], [0]],
                          preferred_element_type=_f32)
    return [wfull[r] for r in range(R)]


def _tc_enc_body(x_ref, wenc_ref, benc_ref, comp_ref, basis_ref, root_ref,
                 yrel_ref, yroot_ref):
    x0 = jnp.dot(x_ref[...], wenc_ref[0], preferred_element_type=_f32)
    x0 = jnp.maximum(x0 + benc_ref[0], 0.0)
    ws = _rel_weights(comp_ref, basis_ref)
    for r in range(R):
        yrel_ref[r] = jnp.dot(x0, ws[r], preferred_element_type=_f32)
    yroot_ref[...] = jnp.dot(x0, root_ref[...], preferred_element_type=_f32)


def _tc_encode_transform(x, wenc, benc, comp, basis, root):
    rows = x.shape[0]
    bm = 1000
    nb = rows // bm
    return pl.pallas_call(
        _tc_enc_body,
        grid=(nb,),
        in_specs=[
            pl.BlockSpec((bm, IN), lambda i: (i, 0)),
            pl.BlockSpec((1, IN, HID), lambda i: (0, 0, 0)),
            pl.BlockSpec((1, 1, HID), lambda i: (0, 0, 0)),
            pl.BlockSpec(comp.shape, lambda i: (0, 0)),
            pl.BlockSpec(basis.shape, lambda i: (0, 0, 0)),
            pl.BlockSpec((HID, HID), lambda i: (0, 0)),
        ],
        out_specs=[
            pl.BlockSpec((R, bm, HID), lambda i: (0, i, 0)),
            pl.BlockSpec((bm, HID), lambda i: (i, 0)),
        ],
        out_shape=[
            jax.ShapeDtypeStruct((R, rows, HID), _f32),
            jax.ShapeDtypeStruct((rows, HID), _f32),
        ],
    )(x, wenc, benc, comp, basis, root)


def _mean_relu(accp_ref, cntp_ref, yroot_ref, bias_ref):
    s = yroot_ref[...] + bias_ref[...]
    for c in range(NC):
        a = accp_ref[c, 0]
        cnt = jnp.max(cntp_ref[c, 0], axis=-1, keepdims=True)
        s = s + a * (1.0 / jnp.maximum(cnt, 1.0))
    a2 = accp_ref[0, 1] + accp_ref[1, 1]
    c2 = jnp.max(cntp_ref[0, 1] + cntp_ref[1, 1], axis=-1, keepdims=True)
    s = s + a2 * (1.0 / jnp.maximum(c2, 1.0))
    return jnp.maximum(s, 0.0)


def _tc_epi_mid_body(accp_ref, cntp_ref, yroot_ref, bias_ref, comp_ref,
                     basis_ref, root_ref, yrel_ref, yroot2_ref):
    x1 = _mean_relu(accp_ref, cntp_ref, yroot_ref, bias_ref)
    ws = _rel_weights(comp_ref, basis_ref)
    for r in range(R):
        yrel_ref[r] = jnp.dot(x1, ws[r], preferred_element_type=_f32)
    yroot2_ref[...] = jnp.dot(x1, root_ref[...], preferred_element_type=_f32)


def _tc_epilogue_mid(accp, cntp, yroot, bias, comp, basis, root):
    bm = 1000
    nb = N // bm
    return pl.pallas_call(
        _tc_epi_mid_body,
        grid=(nb,),
        in_specs=[
            pl.BlockSpec((NC, 2, bm, HID), lambda i: (0, 0, i, 0)),
            pl.BlockSpec((NC, 2, bm, 16), lambda i: (0, 0, i, 0)),
            pl.BlockSpec((bm, HID), lambda i: (i, 0)),
            pl.BlockSpec((1, HID), lambda i: (0, 0)),
            pl.BlockSpec(comp.shape, lambda i: (0, 0)),
            pl.BlockSpec(basis.shape, lambda i: (0, 0, 0)),
            pl.BlockSpec((HID, HID), lambda i: (0, 0)),
        ],
        out_specs=[
            pl.BlockSpec((R, bm, HID), lambda i: (0, i, 0)),
            pl.BlockSpec((bm, HID), lambda i: (i, 0)),
        ],
        out_shape=[
            jax.ShapeDtypeStruct((R, N, HID), _f32),
            jax.ShapeDtypeStruct((N, HID), _f32),
        ],
    )(accp, cntp, yroot, bias.reshape(1, HID), comp, basis, root)


def _tc_epi_fin_body(accp_ref, cntp_ref, yroot_ref, bias_ref, wro_ref,
                     bro_ref, out_ref):
    x2 = _mean_relu(accp_ref, cntp_ref, yroot_ref, bias_ref)
    out_ref[...] = jnp.dot(x2, wro_ref[...], preferred_element_type=_f32) \
        + bro_ref[...]


def _tc_epilogue_final(accp, cntp, yroot, bias, wro, bro):
    bm = 1000
    nb = NF // bm
    return pl.pallas_call(
        _tc_epi_fin_body,
        grid=(nb,),
        in_specs=[
            pl.BlockSpec((NC, 2, bm, HID), lambda i: (0, 0, i, 0)),
            pl.BlockSpec((NC, 2, bm, 16), lambda i: (0, 0, i, 0)),
            pl.BlockSpec((bm, HID), lambda i: (i, 0)),
            pl.BlockSpec((1, HID), lambda i: (0, 0)),
            pl.BlockSpec((HID, 1), lambda i: (0, 0)),
            pl.BlockSpec((1, 1), lambda i: (0, 0)),
        ],
        out_specs=pl.BlockSpec((bm, 1), lambda i: (i, 0)),
        out_shape=jax.ShapeDtypeStruct((NF, 1), _f32),
    )(accp, cntp, yroot, bias.reshape(1, HID), wro, bro.reshape(1, 1))


def kernel(x_flight, x_airport, edge_index_fa, edge_index_af, edge_index_ff,
           W_enc_flight, b_enc_flight, W_enc_airport, b_enc_airport,
           comp0, basis0, root0, bias0, comp1, basis1, root1, bias1,
           W_ro, b_ro):
    fa0 = edge_index_fa[0].astype(jnp.int32)
    fa1 = edge_index_fa[1].astype(jnp.int32)
    af0 = edge_index_af[0].astype(jnp.int32)
    af1 = edge_index_af[1].astype(jnp.int32)
    ff0 = edge_index_ff[0].astype(jnp.int32)
    ff1 = edge_index_ff[1].astype(jnp.int32)
    eh = ff0.shape[0] // 2
    npad = EPC - (fa0.shape[0] + eh)
    padi = jnp.zeros((npad,), jnp.int32)
    padd = jnp.full((npad,), DUMMY, jnp.int32)
    src_c0 = jnp.concatenate([fa0, ff0[:eh] + 2 * N, padi])
    dst_c0 = jnp.concatenate([fa1 + NF, ff1[:eh] + N, padd])
    src_c1 = jnp.concatenate([af0 + (NF + N), ff0[eh:] + 2 * N, padi])
    dst_c1 = jnp.concatenate([af1, ff1[eh:] + N, padd])
    src3 = jnp.concatenate([src_c0, src_c1]).reshape(NW, NCHUNK, CHUNK)
    dst3 = jnp.concatenate([dst_c0, dst_c1]).reshape(NW, NCHUNK, CHUNK)

    zrows = jnp.zeros((RPW, HID), _f32)
    z16 = jnp.zeros((RPW, 16), _f32)
    ones16 = jnp.ones((CHUNK, 16), _f32)

    cnt_out = _sc_count_kernel(dst3, ones16, z16)
    cntp = cnt_out[:, :2 * N, :].reshape(NC, 2, N, 16)

    yrel_f, yroot_f = _tc_encode_transform(
        x_flight, W_enc_flight.reshape(1, IN, HID),
        b_enc_flight.reshape(1, 1, HID), comp0, basis0, root0)
    yrel_a, yroot_a = _tc_encode_transform(
        x_airport, W_enc_airport.reshape(1, IN, HID),
        b_enc_airport.reshape(1, 1, HID), comp0, basis0, root0)
    yrel0 = jnp.concatenate([yrel_f, yrel_a], axis=1)
    yroot0 = jnp.concatenate([yroot_f, yroot_a], axis=0)
    yflat0 = yrel0.reshape(R * N, HID)

    acc0 = _sc_edge_kernel(yflat0, src3, dst3, zrows)
    accp0 = acc0[:, :2 * N, :].reshape(NC, 2, N, HID)
    yrel1, yroot1 = _tc_epilogue_mid(
        accp0, cntp, yroot0, bias0, comp1, basis1, root1)
    yflat1 = yrel1.reshape(R * N, HID)

    acc1 = _sc_edge_kernel(yflat1, src3, dst3, zrows)
    accp1 = acc1[:, :2 * N, :].reshape(NC, 2, N, HID)
    out = _tc_epilogue_final(accp1, cntp, yroot1[:NF], bias1, W_ro, b_ro)
    return out.squeeze(-1)
